# Initial kernel scaffold; baseline (speedup 1.0000x reference)
import jax
import jax.numpy as jnp
from jax.experimental import pallas as pl


def kernel(x, Wr, br, W1, b1, W2, b2):
    raise NotImplementedError("write your pallas kernel here")


# speedup vs baseline: 3.1280x; 3.1280x over previous
import functools
import math

import jax
import jax.numpy as jnp
from jax.experimental import pallas as pl
from jax.experimental.pallas import tpu as pltpu

_INV_SQRT2 = 1.0 / math.sqrt(2.0)


def _moe_body(x_ref, wr_ref, br_ref, w1_ref, b1_ref, w2_ref, b2_ref,
              out_ref, gates_ref, acc_ref, *, T, E):
    e = pl.program_id(0)
    tb = pl.program_id(1)
    xb = x_ref[...]

    @pl.when(e == 0)
    def _():
        logits = jnp.dot(xb, wr_ref[...], preferred_element_type=jnp.float32)
        logits = logits + br_ref[0]
        m = jnp.max(logits, axis=-1, keepdims=True)
        ex = jnp.exp(logits - m)
        p = ex / jnp.sum(ex, axis=-1, keepdims=True)
        cols = jax.lax.broadcasted_iota(jnp.int32, p.shape, 1)
        m1 = jnp.max(p, axis=-1, keepdims=True)
        i1 = jnp.min(jnp.where(p >= m1, cols, E), axis=-1, keepdims=True)
        p2 = jnp.where(cols == i1, -1.0, p)
        m2 = jnp.max(p2, axis=-1, keepdims=True)
        i2 = jnp.min(jnp.where(p2 >= m2, cols, E), axis=-1, keepdims=True)
        g = jnp.where(cols == i1, m1, 0.0) + jnp.where(cols == i2, m2, 0.0)
        gates_ref[pl.ds(tb * T, T), :] = g / (m1 + m2)

    h = jnp.dot(xb, w1_ref[0], preferred_element_type=jnp.float32) + b1_ref[0]
    a = 0.5 * h * (1.0 + jax.lax.erf(h * _INV_SQRT2))
    y = jnp.dot(a, w2_ref[0], preferred_element_type=jnp.float32) + b2_ref[0]

    gblk = gates_ref[pl.ds(tb * T, T), :]
    cols = jax.lax.broadcasted_iota(jnp.int32, gblk.shape, 1)
    g_e = jnp.sum(jnp.where(cols == e, gblk, 0.0), axis=-1, keepdims=True)
    contrib = (y * g_e)[None]

    @pl.when(e == 0)
    def _():
        acc_ref[pl.ds(tb, 1)] = contrib

    @pl.when(e != 0)
    def _():
        acc_ref[pl.ds(tb, 1)] += contrib

    @pl.when(e == E - 1)
    def _():
        out_ref[...] = acc_ref[pl.ds(tb, 1)][0]


@jax.jit
def kernel(x, Wr, br, W1, b1, W2, b2):
    B, S, D = x.shape
    E = Wr.shape[1]
    F = W1.shape[2]
    N = B * S
    T = 256
    TB = N // T
    assert N % T == 0

    xf = x.reshape(N, D)
    br2 = br.reshape(1, E)
    b1r = b1.reshape(E, 1, F)
    b2r = b2.reshape(E, 1, D)

    out = pl.pallas_call(
        functools.partial(_moe_body, T=T, E=E),
        grid=(E, TB),
        in_specs=[
            pl.BlockSpec((T, D), lambda e, tb: (tb, 0)),
            pl.BlockSpec((D, E), lambda e, tb: (0, 0)),
            pl.BlockSpec((1, E), lambda e, tb: (0, 0)),
            pl.BlockSpec((1, D, F), lambda e, tb: (e, 0, 0)),
            pl.BlockSpec((1, 1, F), lambda e, tb: (e, 0, 0)),
            pl.BlockSpec((1, F, D), lambda e, tb: (e, 0, 0)),
            pl.BlockSpec((1, 1, D), lambda e, tb: (e, 0, 0)),
        ],
        out_specs=pl.BlockSpec((T, D), lambda e, tb: (tb, 0)),
        out_shape=jax.ShapeDtypeStruct((N, D), jnp.float32),
        scratch_shapes=[
            pltpu.VMEM((N, E), jnp.float32),
            pltpu.VMEM((TB, T, D), jnp.float32),
        ],
    )(xf, Wr, br2, W1, b1r, W2, b2r)
    return out.reshape(B, S, D)
